# Initial kernel scaffold; baseline (speedup 1.0000x reference)
import jax
import jax.numpy as jnp
from jax.experimental import pallas as pl


def kernel(x, v_ls):
    raise NotImplementedError("write your pallas kernel here")


# speedup vs baseline: 1.8684x; 1.8684x over previous
import jax
import jax.numpy as jnp
from jax.experimental import pallas as pl

B, D, Fd, T = 8, 768, 32, 128
C = 2
D2 = D // C
HW = 196
BD = 128
NJ = D // BD


def _pool_copy_body(x_ref, pooled_ref, out_ref):
    xb = x_ref[...]
    pooled_ref[...] = jnp.max(xb, axis=(2, 3)).reshape(1, 1, 1, BD)
    out_ref[...] = xb


def _bcast_body(prev_ref, sel_ref, out_ref):
    s = sel_ref[...]
    out_ref[...] = jnp.broadcast_to(s.reshape(1, BD, 1, 1), (1, BD, Fd, T))


def _middle_jnp(pooled, v):
    eps = 1e-8
    p = pooled.reshape(B, C, D2)
    na = jnp.sqrt(jnp.sum(p * p, axis=-1))
    nb = jnp.sqrt(jnp.sum(v * v, axis=1))
    dots = jnp.einsum("bcd,bdh->bch", p, v)
    maps = dots / (jnp.maximum(na, eps)[:, :, None] * jnp.maximum(nb, eps)[:, None, :])
    max_ind = jnp.argmax(maps, axis=-1)
    sel = jnp.take_along_axis(v[:, None], max_ind[:, :, None, None], axis=3)[..., 0]
    scores = -jnp.max(maps, axis=-1)
    match_loss = (
        jnp.sum(scores, axis=-1).mean().reshape(1)
        + maps.sum(-1).sum(-1).mean().reshape(1) / HW
    )
    s0, s1 = sel[:, 0], sel[:, 1]
    d01 = jnp.sum(s0 * s1, axis=-1)
    n0 = jnp.sqrt(jnp.sum(s0 * s0, axis=-1))
    n1 = jnp.sqrt(jnp.sum(s1 * s1, axis=-1))
    penalty = (d01 / (jnp.maximum(n0, eps) * jnp.maximum(n1, eps))).mean().reshape(1)
    selflat = sel.reshape(B, D)
    return selflat, maps, match_loss + penalty


def kernel(x, v_ls):
    v = v_ls[0].reshape(B, D2, HW)

    pooled, out1 = pl.pallas_call(
        _pool_copy_body,
        grid=(B, NJ),
        in_specs=[pl.BlockSpec((1, BD, Fd, T), lambda b, j: (b, j, 0, 0))],
        out_specs=[
            pl.BlockSpec((1, 1, 1, BD), lambda b, j: (b, j, 0, 0)),
            pl.BlockSpec((1, BD, Fd, T), lambda b, j: (b, NJ + j, 0, 0)),
        ],
        out_shape=[
            jax.ShapeDtypeStruct((B, NJ, 1, BD), jnp.float32),
            jax.ShapeDtypeStruct((B, 2 * D, Fd, T), jnp.float32),
        ],
    )(x)
    pooled = pooled.reshape(B, D)

    selflat, maps, match_loss = _middle_jnp(pooled, v)

    out = pl.pallas_call(
        _bcast_body,
        grid=(B, NJ),
        in_specs=[
            pl.BlockSpec((1, 1, 8, 128), lambda b, j: (0, 0, 0, 0)),
            pl.BlockSpec((1, 1, 1, BD), lambda b, j: (b, j, 0, 0)),
        ],
        out_specs=pl.BlockSpec((1, BD, Fd, T), lambda b, j: (b, j, 0, 0)),
        out_shape=jax.ShapeDtypeStruct((B, 2 * D, Fd, T), jnp.float32),
        input_output_aliases={0: 0},
    )(out1, selflat.reshape(B, NJ, 1, BD))

    return out, match_loss, maps.reshape(B, C, 14, 14)
